# baseline (speedup 1.0000x reference)
import jax
import jax.numpy as jnp
from jax.experimental import pallas as pl
from jax.experimental.pallas import tpu as pltpu

_LANE = 128
_CAP = 64
_RAMP0 = 4


def _schedule(total):
    segs, sz, rem = [], _RAMP0, total
    while rem > 0:
        s = min(sz, _CAP, rem)
        segs.append(s)
        rem -= s
        sz *= 2
    return segs


def _make_pipeline_kernel(segs, F):
    starts = []
    acc = 0
    for s in segs:
        starts.append(acc)
        acc += s

    def body(b_ref, x_hbm, w_ref, o_ref, x_buf, sems):
        n = len(segs)
        copies = [None] * n

        def start(i):
            st, sz = starts[i], segs[i]
            slot = i % 2
            copies[i] = pltpu.make_async_copy(
                x_hbm.at[pl.ds(st, sz)],
                x_buf.at[slot, pl.ds(0, sz)],
                sems.at[slot],
            )
            copies[i].start()

        start(0)
        w_lane = w_ref[...][:, 0].reshape(1, 1, F)
        bias = b_ref[0, 0]
        for i in range(n):
            if i + 1 < n:
                start(i + 1)
            copies[i].wait()
            st, sz = starts[i], segs[i]
            z = x_buf[i % 2, :sz] * w_lane
            o_ref[pl.ds(st, sz), :] = jnp.sum(z, axis=2) + bias

    return body


def kernel(x, wt_padded, b_padded):
    B, F = x.shape
    dtype = x.dtype

    n_rows = B
    pad = (-n_rows) % _LANE
    if pad:
        x = jnp.pad(x, ((0, pad), (0, 0)))
        B = x.shape[0]

    s_total = B // _LANE
    x3 = x.reshape(s_total, _LANE, F)
    n_pad = wt_padded.shape[1]

    segs = _schedule(s_total)
    cap = max(segs)

    out = pl.pallas_call(
        _make_pipeline_kernel(segs, F),
        out_shape=jax.ShapeDtypeStruct((s_total, _LANE), dtype),
        in_specs=[
            pl.BlockSpec(memory_space=pltpu.SMEM),
            pl.BlockSpec(memory_space=pl.ANY),
            pl.BlockSpec(memory_space=pltpu.VMEM),
        ],
        out_specs=pl.BlockSpec(memory_space=pltpu.VMEM),
        scratch_shapes=[
            pltpu.VMEM((2, cap, _LANE, F), dtype),
            pltpu.SemaphoreType.DMA((2,)),
        ],
        cost_estimate=pl.CostEstimate(
            flops=2 * B * F,
            transcendentals=0,
            bytes_accessed=B * F * 4 + F * n_pad * 4 + B * 4,
        ),
    )(b_padded, x3, wt_padded)

    return out.reshape(B, 1)[:n_rows]


# speedup vs baseline: 1.0571x; 1.0571x over previous
import jax
import jax.numpy as jnp
from jax.experimental import pallas as pl
from jax.experimental.pallas import tpu as pltpu

_LANE = 128


def _rowdot_kernel(b_ref, x_ref, w_ref, o_ref):
    w_lane = w_ref[...][:, 0].reshape(1, 1, w_ref.shape[0])
    z = x_ref[...] * w_lane
    o_ref[...] = jnp.sum(z, axis=2) + b_ref[0, 0]


def _pick_block(n, candidates):
    for c in candidates:
        if n % c == 0:
            return c
    return 1


def kernel(x, wt_padded, b_padded):
    B, F = x.shape
    dtype = x.dtype

    n_rows = B
    pad = (-n_rows) % _LANE
    if pad:
        x = jnp.pad(x, ((0, pad), (0, 0)))
        B = x.shape[0]

    s_total = B // _LANE
    x3 = x.reshape(s_total, _LANE, F)
    n_pad = wt_padded.shape[1]

    s_blk = 48 if s_total % 48 == 32 else _pick_block(s_total, (64, 32, 16, 8, 4, 2, 1))
    grid = ((s_total + s_blk - 1) // s_blk,)

    out = pl.pallas_call(
        _rowdot_kernel,
        out_shape=jax.ShapeDtypeStruct((s_total, _LANE), dtype),
        grid_spec=pl.GridSpec(
            grid=grid,
            in_specs=[
                pl.BlockSpec(memory_space=pltpu.SMEM),
                pl.BlockSpec((s_blk, _LANE, F), lambda i: (i, 0, 0)),
                pl.BlockSpec((F, n_pad), lambda i: (0, 0)),
            ],
            out_specs=pl.BlockSpec((s_blk, _LANE), lambda i: (i, 0)),
        ),
        compiler_params=pltpu.CompilerParams(
            dimension_semantics=("arbitrary",),
        ),
        cost_estimate=pl.CostEstimate(
            flops=2 * B * F,
            transcendentals=0,
            bytes_accessed=B * F * 4 + F * n_pad * 4 + B * 4,
        ),
    )(b_padded, x3, wt_padded)

    return out.reshape(B, 1)[:n_rows]


# speedup vs baseline: 1.1069x; 1.0471x over previous
import jax
import jax.numpy as jnp
from jax.experimental import pallas as pl
from jax.experimental.pallas import tpu as pltpu

_LANE = 128


def _rowdot_kernel(b_ref, x_ref, w_ref, o_ref):
    w_lane = w_ref[...][:, 0].reshape(1, 1, w_ref.shape[0])
    z = x_ref[...] * w_lane
    o_ref[...] = jnp.sum(z, axis=2) + b_ref[0, 0]


def _pick_block(n, candidates):
    for c in candidates:
        if n % c == 0:
            return c
    return 1


def kernel(x, wt_padded, b_padded):
    B, F = x.shape
    dtype = x.dtype

    n_rows = B
    pad = (-n_rows) % _LANE
    if pad:
        x = jnp.pad(x, ((0, pad), (0, 0)))
        B = x.shape[0]

    s_total = B // _LANE
    x3 = x.reshape(s_total, _LANE, F)
    n_pad = wt_padded.shape[1]

    s_blk = _pick_block(s_total, (64, 32, 16, 8, 4, 2, 1))
    grid = (s_total // s_blk,)

    out = pl.pallas_call(
        _rowdot_kernel,
        out_shape=jax.ShapeDtypeStruct((s_total, _LANE), dtype),
        grid_spec=pl.GridSpec(
            grid=grid,
            in_specs=[
                pl.BlockSpec(memory_space=pltpu.SMEM),
                pl.BlockSpec((s_blk, _LANE, F), lambda i: (i, 0, 0)),
                pl.BlockSpec((F, n_pad), lambda i: (0, 0)),
            ],
            out_specs=pl.BlockSpec((s_blk, _LANE), lambda i: (i, 0)),
        ),
        compiler_params=pltpu.CompilerParams(
            dimension_semantics=("arbitrary",),
        ),
        cost_estimate=pl.CostEstimate(
            flops=2 * B * F,
            transcendentals=0,
            bytes_accessed=B * F * 4 + F * n_pad * 4 + B * 4,
        ),
    )(b_padded, x3, wt_padded)

    return out.reshape(B, 1)[:n_rows]
